# baseline (speedup 1.0000x reference)
import numpy as np

import jax
import jax.numpy as jnp
from jax import lax
from jax.experimental import pallas as pl
from jax.experimental.pallas import tpu as pltpu

_TOP_K = 2
_NEG_INF = float("-inf")

_noise_cache = {}


def _noise_const(bs, ne):
    if (bs, ne) not in _noise_cache:
        with jax.ensure_compile_time_eval():
            arr = jax.random.normal(jax.random.key(42), (bs, ne), jnp.float32)
        _noise_cache[(bs, ne)] = np.asarray(arr)
    return _noise_cache[(bs, ne)]


def _router_body(xc_ref, wr_ref, br_ref, wn_ref, bn_ref, noise_ref,
                 idx_ref, gate_ref, pooled_ref):
    b = pl.program_id(0)
    nsteps = pl.num_programs(0)
    pooled_ref[pl.ds(b, 1), :] = jnp.mean(xc_ref[0], axis=1)[None, :]

    @pl.when(b == nsteps - 1)
    def _():
        pooled = pooled_ref[...]
        logits = jnp.dot(pooled, wr_ref[...],
                         preferred_element_type=jnp.float32) + br_ref[0]
        nlog = jnp.dot(pooled, wn_ref[...],
                       preferred_element_type=jnp.float32) + bn_ref[0]
        noisy = logits + noise_ref[...] * jax.nn.softplus(nlog)

        bs, ne = noisy.shape
        eids = lax.broadcasted_iota(jnp.int32, (bs, ne), 1)
        v0 = jnp.max(noisy, axis=1)
        i0 = jnp.min(jnp.where(noisy == v0[:, None], eids, ne), axis=1)
        masked = jnp.where(eids == i0[:, None], _NEG_INF, noisy)
        v1 = jnp.max(masked, axis=1)
        i1 = jnp.min(jnp.where(masked == v1[:, None], eids, ne), axis=1)
        t = jnp.exp(v1 - v0)
        g0 = 1.0 / (1.0 + t)
        g1 = t / (1.0 + t)
        idx_ref[...] = jnp.concatenate([i0[None, :], i1[None, :]], axis=0)
        gate_ref[...] = jnp.concatenate([g0[None, :], g1[None, :]], axis=0)


def _expert_body(idx_ref, gate_ref, xc_ref, w1a_ref, w1b_ref, b1a_ref,
                 b1b_ref, w2a_ref, w2b_ref, b2a_ref, b2b_ref, out_ref):
    b = pl.program_id(0)
    g0 = gate_ref[0, b]
    g1 = gate_ref[1, b]
    xb = xc_ref[0]
    cdim = (((0,), (0,)), ((), ()))
    h1a = jnp.maximum(
        lax.dot_general(w1a_ref[0], xb, cdim,
                        preferred_element_type=jnp.float32)
        + b1a_ref[0], 0.0)
    h1b = jnp.maximum(
        lax.dot_general(w1b_ref[0], xb, cdim,
                        preferred_element_type=jnp.float32)
        + b1b_ref[0], 0.0)
    h2a = lax.dot_general(w2a_ref[0], h1a, cdim,
                          preferred_element_type=jnp.float32)
    h2b = lax.dot_general(w2b_ref[0], h1b, cdim,
                          preferred_element_type=jnp.float32)
    out_ref[0] = g0 * (h2a + b2a_ref[0]) + g1 * (h2b + b2b_ref[0])


def kernel(x, Wr, br, Wn, bn, W1, b1, W2, b2):
    bs, dim, h, w = x.shape
    hw = h * w
    ne = Wr.shape[1]
    hid = W1.shape[2]

    xc = x.reshape(bs, dim, hw)
    noise = jnp.asarray(_noise_const(bs, ne))

    idx, gates = pl.pallas_call(
        _router_body,
        grid=(bs,),
        in_specs=[
            pl.BlockSpec((1, dim, hw), lambda b: (b, 0, 0)),
            pl.BlockSpec((dim, ne), lambda b: (0, 0)),
            pl.BlockSpec((1, ne), lambda b: (0, 0)),
            pl.BlockSpec((dim, ne), lambda b: (0, 0)),
            pl.BlockSpec((1, ne), lambda b: (0, 0)),
            pl.BlockSpec((bs, ne), lambda b: (0, 0)),
        ],
        out_specs=(
            pl.BlockSpec((_TOP_K, bs), lambda b: (0, 0)),
            pl.BlockSpec((_TOP_K, bs), lambda b: (0, 0)),
        ),
        out_shape=(
            jax.ShapeDtypeStruct((_TOP_K, bs), jnp.int32),
            jax.ShapeDtypeStruct((_TOP_K, bs), jnp.float32),
        ),
        scratch_shapes=[pltpu.VMEM((bs, dim), jnp.float32)],
    )(xc, Wr, br.reshape(1, ne), Wn, bn.reshape(1, ne), noise)

    def _e0(b, i_ref, g_ref):
        return (i_ref[0, b], 0, 0)

    def _e1(b, i_ref, g_ref):
        return (i_ref[1, b], 0, 0)

    grid_spec = pltpu.PrefetchScalarGridSpec(
        num_scalar_prefetch=2,
        grid=(bs,),
        in_specs=[
            pl.BlockSpec((1, dim, hw), lambda b, i_ref, g_ref: (b, 0, 0)),
            pl.BlockSpec((1, dim, hid), _e0),
            pl.BlockSpec((1, dim, hid), _e1),
            pl.BlockSpec((1, hid, 1), _e0),
            pl.BlockSpec((1, hid, 1), _e1),
            pl.BlockSpec((1, hid, dim), _e0),
            pl.BlockSpec((1, hid, dim), _e1),
            pl.BlockSpec((1, dim, 1), _e0),
            pl.BlockSpec((1, dim, 1), _e1),
        ],
        out_specs=pl.BlockSpec((1, dim, hw), lambda b, i_ref, g_ref: (b, 0, 0)),
    )
    outp = pl.pallas_call(
        _expert_body,
        grid_spec=grid_spec,
        out_shape=jax.ShapeDtypeStruct((bs, dim, hw), jnp.float32),
    )(idx, gates, xc, W1, W1, b1.reshape(ne, hid, 1), b1.reshape(ne, hid, 1),
      W2, W2, b2.reshape(ne, dim, 1), b2.reshape(ne, dim, 1))

    return outp.reshape(bs, dim, h, w)


# speedup vs baseline: 1.1445x; 1.1445x over previous
import numpy as np

import jax
import jax.numpy as jnp
from jax import lax
from jax.experimental import pallas as pl
from jax.experimental.pallas import tpu as pltpu

_TOP_K = 2
_NEG_INF = float("-inf")

_noise_cache = {}


def _noise_const(bs, ne):
    if (bs, ne) not in _noise_cache:
        with jax.ensure_compile_time_eval():
            arr = jax.random.normal(jax.random.key(42), (bs, ne), jnp.float32)
        _noise_cache[(bs, ne)] = np.asarray(arr)
    return _noise_cache[(bs, ne)]


def _router_body(xc_ref, wr_ref, br_ref, wn_ref, bn_ref, noise_ref,
                 idx_ref, gate_ref, pooled_ref):
    b = pl.program_id(0)
    nsteps = pl.num_programs(0)
    pooled_ref[pl.ds(b, 1), :] = jnp.mean(xc_ref[0], axis=1)[None, :]

    @pl.when(b == nsteps - 1)
    def _():
        pooled = pooled_ref[...]
        logits = jnp.dot(pooled, wr_ref[...],
                         preferred_element_type=jnp.float32) + br_ref[0]
        nlog = jnp.dot(pooled, wn_ref[...],
                       preferred_element_type=jnp.float32) + bn_ref[0]
        noisy = logits + noise_ref[...] * jax.nn.softplus(nlog)

        bs, ne = noisy.shape
        eids = lax.broadcasted_iota(jnp.int32, (bs, ne), 1)
        v0 = jnp.max(noisy, axis=1)
        i0 = jnp.min(jnp.where(noisy == v0[:, None], eids, ne), axis=1)
        masked = jnp.where(eids == i0[:, None], _NEG_INF, noisy)
        v1 = jnp.max(masked, axis=1)
        i1 = jnp.min(jnp.where(masked == v1[:, None], eids, ne), axis=1)
        t = jnp.exp(v1 - v0)
        g0 = 1.0 / (1.0 + t)
        g1 = t / (1.0 + t)
        idx_ref[...] = jnp.concatenate([i0[None, :], i1[None, :]], axis=0)
        gate_ref[...] = jnp.concatenate([g0[None, :], g1[None, :]], axis=0)


def _expert_body(idx_ref, gate_ref, xc_ref, w1a_ref, w1b_ref, b1_ref,
                 w2a_ref, w2b_ref, b2_ref, out_ref):
    b = pl.program_id(0)
    e0 = idx_ref[0, b]
    e1 = idx_ref[1, b]
    g0 = gate_ref[0, b]
    g1 = gate_ref[1, b]
    hid = b1_ref.shape[1]
    dim = b2_ref.shape[1]
    b1a = b1_ref[pl.ds(e0, 1), :].reshape(hid, 1)
    b1b = b1_ref[pl.ds(e1, 1), :].reshape(hid, 1)
    b2a = b2_ref[pl.ds(e0, 1), :].reshape(dim, 1)
    b2b = b2_ref[pl.ds(e1, 1), :].reshape(dim, 1)
    xb = xc_ref[0]
    cdim = (((0,), (0,)), ((), ()))
    h1a = jnp.maximum(
        lax.dot_general(w1a_ref[0], xb, cdim,
                        preferred_element_type=jnp.float32)
        + b1a, 0.0)
    h1b = jnp.maximum(
        lax.dot_general(w1b_ref[0], xb, cdim,
                        preferred_element_type=jnp.float32)
        + b1b, 0.0)
    h2a = lax.dot_general(w2a_ref[0], h1a, cdim,
                          preferred_element_type=jnp.float32)
    h2b = lax.dot_general(w2b_ref[0], h1b, cdim,
                          preferred_element_type=jnp.float32)
    out_ref[0] = g0 * (h2a + b2a) + g1 * (h2b + b2b)


def kernel(x, Wr, br, Wn, bn, W1, b1, W2, b2):
    bs, dim, h, w = x.shape
    hw = h * w
    ne = Wr.shape[1]
    hid = W1.shape[2]

    xc = x.reshape(bs, dim, hw)
    noise = jnp.asarray(_noise_const(bs, ne))

    idx, gates = pl.pallas_call(
        _router_body,
        grid=(bs,),
        in_specs=[
            pl.BlockSpec((1, dim, hw), lambda b: (b, 0, 0)),
            pl.BlockSpec((dim, ne), lambda b: (0, 0)),
            pl.BlockSpec((1, ne), lambda b: (0, 0)),
            pl.BlockSpec((dim, ne), lambda b: (0, 0)),
            pl.BlockSpec((1, ne), lambda b: (0, 0)),
            pl.BlockSpec((bs, ne), lambda b: (0, 0)),
        ],
        out_specs=(
            pl.BlockSpec((_TOP_K, bs), lambda b: (0, 0)),
            pl.BlockSpec((_TOP_K, bs), lambda b: (0, 0)),
        ),
        out_shape=(
            jax.ShapeDtypeStruct((_TOP_K, bs), jnp.int32),
            jax.ShapeDtypeStruct((_TOP_K, bs), jnp.float32),
        ),
        scratch_shapes=[pltpu.VMEM((bs, dim), jnp.float32)],
    )(xc, Wr, br.reshape(1, ne), Wn, bn.reshape(1, ne), noise)

    def _e0(b, i_ref, g_ref):
        return (i_ref[0, b], 0, 0)

    def _e1(b, i_ref, g_ref):
        return (i_ref[1, b], 0, 0)

    grid_spec = pltpu.PrefetchScalarGridSpec(
        num_scalar_prefetch=2,
        grid=(bs,),
        in_specs=[
            pl.BlockSpec((1, dim, hw), lambda b, i_ref, g_ref: (b, 0, 0)),
            pl.BlockSpec((1, dim, hid), _e0),
            pl.BlockSpec((1, dim, hid), _e1),
            pl.BlockSpec((ne, hid), lambda b, i_ref, g_ref: (0, 0)),
            pl.BlockSpec((1, hid, dim), _e0),
            pl.BlockSpec((1, hid, dim), _e1),
            pl.BlockSpec((ne, dim), lambda b, i_ref, g_ref: (0, 0)),
        ],
        out_specs=pl.BlockSpec((1, dim, hw), lambda b, i_ref, g_ref: (b, 0, 0)),
    )
    outp = pl.pallas_call(
        _expert_body,
        grid_spec=grid_spec,
        out_shape=jax.ShapeDtypeStruct((bs, dim, hw), jnp.float32),
    )(idx, gates, xc, W1, W1, b1, W2, W2, b2)

    return outp.reshape(bs, dim, h, w)
